# baseline (speedup 1.0000x reference)
import functools

import jax
import jax.numpy as jnp
from jax import lax
from jax.experimental import pallas as pl
from jax.experimental.pallas import tpu as pltpu
from jax.experimental.pallas import tpu_sc as plsc

BATCH = 16384
NUM_FIELDS = 26
FIELD_SIZE = 100000

NC = 2
NS = 16
NW = NC * NS
B_PER_W = BATCH // NW
N_IDX = B_PER_W * NUM_FIELDS


RROWS = 125
RVALS = RROWS * 64
N_RCH = 40625 // RROWS


def _reformat_body(tr_ref, flat_ref, rub0, rub1, fv0, fv1, si, so):
    wid = lax.axis_index("s") * NC + lax.axis_index("c")
    cnt = 10 + (wid < N_RCH % NW).astype(jnp.int32)

    def start_in(c, rb):
        g = wid + c * NW
        pltpu.async_copy(tr_ref.at[pl.ds(g * RROWS, RROWS), :], rb, si)

    def wait_in(c, rb):
        g = wid + c * NW
        pltpu.make_async_copy(
            tr_ref.at[pl.ds(g * RROWS, RROWS), :], rb, si
        ).wait()

    def process(rb, fv):
        def step(i, _):
            r = i // 4
            col = (i % 4) * 16
            fv[pl.ds(i * 16, 16)] = rb[r, pl.ds(col, 16)]
            return 0

        lax.fori_loop(0, RVALS // 16, step, 0, unroll=8)

    def start_out(c, fv):
        g = wid + c * NW
        pltpu.async_copy(fv, flat_ref.at[pl.ds(g * RVALS, RVALS)], so)

    def wait_out(c, fv):
        g = wid + c * NW
        pltpu.make_async_copy(
            fv, flat_ref.at[pl.ds(g * RVALS, RVALS)], so
        ).wait()

    start_in(0, rub0)

    def pair(i2, _):
        c0 = 2 * i2
        c1 = c0 + 1

        @pl.when(c1 < cnt)
        def _():
            start_in(c1, rub1)

        wait_in(c0, rub0)

        @pl.when(i2 > 0)
        def _():
            wait_out(c0 - 2, fv0)

        process(rub0, fv0)
        start_out(c0, fv0)

        @pl.when(c0 + 2 < cnt)
        def _():
            start_in(c0 + 2, rub0)

        @pl.when(c1 < cnt)
        def _():
            wait_in(c1, rub1)

            @pl.when(i2 > 0)
            def _():
                wait_out(c1 - 2, fv1)

            process(rub1, fv1)
            start_out(c1, fv1)

        return 0

    npairs = (cnt + 1) // 2
    lax.fori_loop(0, npairs, pair, 0)

    wait_out(2 * ((cnt - 1) // 2), fv0)
    wait_out(2 * ((cnt - 2) // 2) + 1, fv1)


def _body(x_ref, table_ref, out_ref, x_v, idx_v, rows_v, out_v, sem):
    wid = lax.axis_index("s") * NC + lax.axis_index("c")
    base = wid * N_IDX

    pltpu.sync_copy(x_ref.at[pl.ds(base, N_IDX)], x_v)

    lanes26 = lax.iota(jnp.int32, 16) * NUM_FIELDS

    def build(t, _):
        f = t // (B_PER_W // 16)
        c2 = t % (B_PER_W // 16)
        xpos = lanes26 + (c2 * 16 * NUM_FIELDS + f)
        xv = plsc.load_gather(x_v, [xpos])
        idx_v[pl.ds(t * 16, 16)] = xv + f * FIELD_SIZE
        return 0

    lax.fori_loop(0, NUM_FIELDS * (B_PER_W // 16), build, 0, unroll=4)

    pltpu.async_copy(table_ref.at[idx_v], rows_v, sem).wait()

    def reduce_chunk(c2, _):
        def add_f(f, acc):
            q = f * B_PER_W + c2 * 16
            return acc + rows_v[pl.ds(q, 16)]

        acc = lax.fori_loop(
            0, NUM_FIELDS, add_f, jnp.zeros((16,), jnp.float32), unroll=4
        )
        out_v[pl.ds(c2 * 16, 16)] = acc
        return 0

    lax.fori_loop(0, B_PER_W // 16, reduce_chunk, 0)

    pltpu.sync_copy(out_v, out_ref.at[pl.ds(wid * B_PER_W, B_PER_W)])


@jax.jit
def kernel(x, table, bias):
    mesh = plsc.VectorSubcoreMesh(core_axis_name="c", subcore_axis_name="s")
    params = pltpu.CompilerParams(
        needs_layout_passes=False, use_tc_tiling_on_sc=False
    )
    reformat = pl.kernel(
        _reformat_body,
        out_type=jax.ShapeDtypeStruct((NUM_FIELDS * FIELD_SIZE,), jnp.float32),
        mesh=mesh,
        compiler_params=params,
        scratch_types=[
            pltpu.VMEM((RROWS, 64), jnp.float32),
            pltpu.VMEM((RROWS, 64), jnp.float32),
            pltpu.VMEM((RVALS,), jnp.float32),
            pltpu.VMEM((RVALS,), jnp.float32),
            pltpu.SemaphoreType.DMA,
            pltpu.SemaphoreType.DMA,
        ],
    )
    k = pl.kernel(
        _body,
        out_type=jax.ShapeDtypeStruct((BATCH,), jnp.float32),
        mesh=mesh,
        compiler_params=params,
        scratch_types=[
            pltpu.VMEM((N_IDX,), jnp.int32),
            pltpu.VMEM((N_IDX,), jnp.int32),
            pltpu.VMEM((N_IDX,), jnp.float32),
            pltpu.VMEM((B_PER_W,), jnp.float32),
            pltpu.SemaphoreType.DMA,
        ],
    )
    flat = reformat(table.reshape(40625, 64))
    out = k(x.reshape(-1), flat)
    return out.reshape(BATCH, 1) + bias[None, :]


# speedup vs baseline: 1.0582x; 1.0582x over previous
import functools

import jax
import jax.numpy as jnp
from jax import lax
from jax.experimental import pallas as pl
from jax.experimental.pallas import tpu as pltpu
from jax.experimental.pallas import tpu_sc as plsc

BATCH = 16384
NUM_FIELDS = 26
FIELD_SIZE = 100000

NC = 2
NS = 16
NW = NC * NS
B_PER_W = BATCH // NW
N_IDX = B_PER_W * NUM_FIELDS


def _build_body(x_ref, idx_ref, x_v, idx_v):
    wid = lax.axis_index("s") * NC + lax.axis_index("c")
    base = wid * B_PER_W

    pltpu.sync_copy(x_ref.at[pl.ds(base, B_PER_W), :], x_v)

    lanes = lax.iota(jnp.int32, 16)

    def build(t, _):
        f = t // (B_PER_W // 16)
        c2 = t % (B_PER_W // 16)
        j16 = c2 * 16 + lanes
        f16 = jnp.zeros((16,), jnp.int32) + f
        xv = plsc.load_gather(x_v, [j16, f16])
        idx_v[pl.ds(t * 16, 16)] = xv + f * FIELD_SIZE
        return 0

    lax.fori_loop(0, NUM_FIELDS * (B_PER_W // 16), build, 0, unroll=4)

    pltpu.sync_copy(idx_v, idx_ref.at[pl.ds(wid * N_IDX, N_IDX)])


def _gather_body(idx_ref, table_ref, out_ref, idx_v, rows_v, out_v, sem):
    wid = lax.axis_index("s") * NC + lax.axis_index("c")

    pltpu.sync_copy(idx_ref.at[pl.ds(wid * N_IDX, N_IDX)], idx_v)

    pltpu.async_copy(table_ref.at[idx_v], rows_v, sem).wait()

    def reduce_chunk(c2, _):
        def add_f(f, acc):
            q = f * B_PER_W + c2 * 16
            return acc + rows_v[pl.ds(q, 16)]

        acc = lax.fori_loop(
            0, NUM_FIELDS, add_f, jnp.zeros((16,), jnp.float32), unroll=4
        )
        out_v[pl.ds(c2 * 16, 16)] = acc
        return 0

    lax.fori_loop(0, B_PER_W // 16, reduce_chunk, 0)

    pltpu.sync_copy(out_v, out_ref.at[pl.ds(wid * B_PER_W, B_PER_W)])


@jax.jit
def kernel(x, table, bias):
    mesh = plsc.VectorSubcoreMesh(core_axis_name="c", subcore_axis_name="s")
    params = pltpu.CompilerParams(
        needs_layout_passes=False, use_tc_tiling_on_sc=False
    )
    build = pl.kernel(
        _build_body,
        out_type=jax.ShapeDtypeStruct((BATCH * NUM_FIELDS,), jnp.int32),
        mesh=mesh,
        compiler_params=params,
        scratch_types=[
            pltpu.VMEM((B_PER_W, NUM_FIELDS), jnp.int32),
            pltpu.VMEM((N_IDX,), jnp.int32),
        ],
    )
    gather = pl.kernel(
        _gather_body,
        out_type=jax.ShapeDtypeStruct((BATCH,), jnp.float32),
        mesh=mesh,
        compiler_params=params,
        scratch_types=[
            pltpu.VMEM((N_IDX,), jnp.int32),
            pltpu.VMEM((N_IDX,), jnp.float32),
            pltpu.VMEM((B_PER_W,), jnp.float32),
            pltpu.SemaphoreType.DMA,
        ],
    )
    idx = build(x)
    out = gather(idx, table.reshape(-1))
    return out.reshape(BATCH, 1) + bias[None, :]


# speedup vs baseline: 1.0594x; 1.0011x over previous
import functools

import jax
import jax.numpy as jnp
from jax import lax
from jax.experimental import pallas as pl
from jax.experimental.pallas import tpu as pltpu
from jax.experimental.pallas import tpu_sc as plsc

BATCH = 16384
NUM_FIELDS = 26
FIELD_SIZE = 100000

NC = 2
NS = 16
NW = NC * NS
B_PER_W = BATCH // NW
N_IDX = B_PER_W * NUM_FIELDS


def _build_body(x_ref, idx_ref, x_v, idx_v):
    wid = lax.axis_index("s") * NC + lax.axis_index("c")
    base = wid * B_PER_W

    pltpu.sync_copy(x_ref.at[pl.ds(base, B_PER_W), :], x_v)

    lanes = lax.iota(jnp.int32, 16)

    def build(t, _):
        f = t // (B_PER_W // 16)
        c2 = t % (B_PER_W // 16)
        j16 = c2 * 16 + lanes
        f16 = jnp.zeros((16,), jnp.int32) + f
        xv = plsc.load_gather(x_v, [j16, f16])
        idx_v[pl.ds(t * 16, 16)] = xv + f * FIELD_SIZE
        return 0

    lax.fori_loop(0, NUM_FIELDS * (B_PER_W // 16), build, 0, unroll=4)

    pltpu.sync_copy(idx_v, idx_ref.at[pl.ds(wid * N_IDX, N_IDX)])


def _gather_body(idx_ref, table_ref, out_ref, idx_v, rows_v, out_v, sem):
    wid = lax.axis_index("s") * NC + lax.axis_index("c")

    pltpu.sync_copy(idx_ref.at[pl.ds(wid * N_IDX, N_IDX)], idx_v)

    pltpu.async_copy(table_ref.at[idx_v], rows_v, sem).wait()

    def reduce_chunk(c2, _):
        def add_f(f, acc):
            q = f * B_PER_W + c2 * 16
            return acc + plsc.bitcast(rows_v[pl.ds(q, 16)], jnp.float32)

        acc = lax.fori_loop(
            0, NUM_FIELDS, add_f, jnp.zeros((16,), jnp.float32), unroll=4
        )
        out_v[pl.ds(c2 * 16, 16)] = acc
        return 0

    lax.fori_loop(0, B_PER_W // 16, reduce_chunk, 0)

    pltpu.sync_copy(out_v, out_ref.at[pl.ds(wid * B_PER_W, B_PER_W)])


@jax.jit
def kernel(x, table, bias):
    mesh = plsc.VectorSubcoreMesh(core_axis_name="c", subcore_axis_name="s")
    params = pltpu.CompilerParams(
        needs_layout_passes=False, use_tc_tiling_on_sc=False
    )
    build = pl.kernel(
        _build_body,
        out_type=jax.ShapeDtypeStruct((BATCH * NUM_FIELDS,), jnp.int32),
        mesh=mesh,
        compiler_params=params,
        scratch_types=[
            pltpu.VMEM((B_PER_W, NUM_FIELDS), jnp.int32),
            pltpu.VMEM((N_IDX,), jnp.int32),
        ],
    )
    gather = pl.kernel(
        _gather_body,
        out_type=jax.ShapeDtypeStruct((BATCH,), jnp.float32),
        mesh=mesh,
        compiler_params=params,
        scratch_types=[
            pltpu.VMEM((N_IDX,), jnp.int32),
            pltpu.VMEM((N_IDX,), jnp.int32),
            pltpu.VMEM((B_PER_W,), jnp.float32),
            pltpu.SemaphoreType.DMA,
        ],
    )
    idx = build(x)
    tab_i = lax.bitcast_convert_type(table, jnp.int32).reshape(-1)
    out = gather(idx, tab_i)
    return out.reshape(BATCH, 1) + bias[None, :]


# speedup vs baseline: 1.0844x; 1.0236x over previous
import functools

import jax
import jax.numpy as jnp
from jax import lax
from jax.experimental import pallas as pl
from jax.experimental.pallas import tpu as pltpu
from jax.experimental.pallas import tpu_sc as plsc

BATCH = 16384
NUM_FIELDS = 26
FIELD_SIZE = 100000

NC = 2
NS = 16
NW = NC * NS
B_PER_W = BATCH // NW
N_IDX = B_PER_W * NUM_FIELDS


def _build_body(x_ref, idx_ref, x_v, idx_v):
    wid = lax.axis_index("s") * NC + lax.axis_index("c")
    base = wid * N_IDX

    pltpu.sync_copy(x_ref.at[pl.ds(base, N_IDX)], x_v)

    lanes26 = lax.iota(jnp.int32, 16) * NUM_FIELDS

    def build(t, _):
        f = t // (B_PER_W // 16)
        c2 = t % (B_PER_W // 16)
        xpos = lanes26 + (c2 * 16 * NUM_FIELDS + f)
        xv = plsc.load_gather(x_v, [xpos])
        idx_v[pl.ds(t * 16, 16)] = xv + f * FIELD_SIZE
        return 0

    lax.fori_loop(0, NUM_FIELDS * (B_PER_W // 16), build, 0, unroll=4)

    pltpu.sync_copy(idx_v, idx_ref.at[pl.ds(wid * N_IDX, N_IDX)])


def _gather_body(idx_ref, table_ref, out_ref, idx_v, rows_v, out_v, sem):
    wid = lax.axis_index("s") * NC + lax.axis_index("c")

    pltpu.sync_copy(idx_ref.at[pl.ds(wid * N_IDX, N_IDX)], idx_v)

    pltpu.async_copy(table_ref.at[idx_v], rows_v, sem).wait()

    def reduce_chunk(c2, _):
        def add_f(f, acc):
            q = f * B_PER_W + c2 * 16
            return acc + plsc.bitcast(rows_v[pl.ds(q, 16)], jnp.float32)

        acc = lax.fori_loop(
            0, NUM_FIELDS, add_f, jnp.zeros((16,), jnp.float32), unroll=4
        )
        out_v[pl.ds(c2 * 16, 16)] = acc
        return 0

    lax.fori_loop(0, B_PER_W // 16, reduce_chunk, 0)

    pltpu.sync_copy(out_v, out_ref.at[pl.ds(wid * B_PER_W, B_PER_W)])


@jax.jit
def kernel(x, table, bias):
    mesh = plsc.VectorSubcoreMesh(core_axis_name="c", subcore_axis_name="s")
    params = pltpu.CompilerParams(
        needs_layout_passes=False, use_tc_tiling_on_sc=False
    )
    build = pl.kernel(
        _build_body,
        out_type=jax.ShapeDtypeStruct((BATCH * NUM_FIELDS,), jnp.int32),
        mesh=mesh,
        compiler_params=params,
        scratch_types=[
            pltpu.VMEM((N_IDX,), jnp.int32),
            pltpu.VMEM((N_IDX,), jnp.int32),
        ],
    )
    gather = pl.kernel(
        _gather_body,
        out_type=jax.ShapeDtypeStruct((BATCH,), jnp.float32),
        mesh=mesh,
        compiler_params=params,
        scratch_types=[
            pltpu.VMEM((N_IDX,), jnp.int32),
            pltpu.VMEM((N_IDX,), jnp.int32),
            pltpu.VMEM((B_PER_W,), jnp.float32),
            pltpu.SemaphoreType.DMA,
        ],
    )
    idx = build(x.reshape(-1))
    tab_i = lax.bitcast_convert_type(table, jnp.int32).reshape(-1)
    out = gather(idx, tab_i)
    return out.reshape(BATCH, 1) + bias[None, :]
